# Initial kernel scaffold; baseline (speedup 1.0000x reference)
import jax
import jax.numpy as jnp
from jax.experimental import pallas as pl


def kernel(x, edge_index, sW, sU, sb, sg, sB, a1W, a1U, a1b, a1g, a1B, b1W, b1U, b1b, b1g, b1B, d1W, d1g, d1B, a2W, a2U, a2b, a2g, a2B, b2W, b2U, b2b, b2g, b2B, d2W, d2g, d2B, lW1, lb1, lW2, lb2):
    raise NotImplementedError("write your pallas kernel here")


# speedup vs baseline: 8.6632x; 8.6632x over previous
import functools

import jax
import jax.numpy as jnp
from jax import lax
from jax.experimental import pallas as pl
from jax.experimental.pallas import tpu as pltpu
from jax.experimental.pallas import tpu_sc as plsc

N = 10000
E = 320000
NCORE = 2
NSUB = 16
NW = NCORE * NSUB
EW = E // NW
CHUNK = 80
NCH = EW // CHUNK
ROW_STRIDE = 624
ROW_SLAB = 640


def _spmm_sc(h, src_r, dst_r, zeros, C):
  mesh = plsc.VectorSubcoreMesh(core_axis_name="c", subcore_axis_name="s")

  @functools.partial(
      pl.kernel,
      out_type=jax.ShapeDtypeStruct((NCORE, N, C), jnp.float32),
      mesh=mesh,
      compiler_params=pltpu.CompilerParams(use_tc_tiling_on_sc=False),
      scratch_types=[
          pltpu.VMEM((NCH, CHUNK), jnp.int32),
          pltpu.VMEM((NCH, CHUNK), jnp.int32),
          pltpu.VMEM((CHUNK, C), jnp.float32),
          pltpu.VMEM_SHARED((N, C), jnp.float32),
          pltpu.SemaphoreType.DMA,
      ],
  )
  def spmm(h_hbm, src_hbm, dst_hbm, z_hbm, out_hbm,
           src_v, dst_v, rows_v, agg_sh, sem):
    cid = lax.axis_index("c")
    sid = lax.axis_index("s")
    wid = sid * NCORE + cid
    r0 = sid * ROW_STRIDE
    pltpu.sync_copy(z_hbm.at[pl.ds(r0, ROW_SLAB)],
                    agg_sh.at[pl.ds(r0, ROW_SLAB)])
    pltpu.sync_copy(src_hbm.at[wid], src_v)
    pltpu.sync_copy(dst_hbm.at[wid], dst_v)
    plsc.subcore_barrier()

    def body(i, carry):
      pltpu.async_copy(h_hbm.at[src_v.at[i]], rows_v, sem).wait()
      pltpu.sync_copy(rows_v, agg_sh.at[dst_v.at[i]], add=True)
      return carry

    lax.fori_loop(0, NCH, body, 0)
    plsc.subcore_barrier()
    pltpu.sync_copy(agg_sh.at[pl.ds(r0, ROW_SLAB)],
                    out_hbm.at[cid, pl.ds(r0, ROW_SLAB)])

  return spmm(h, src_r, dst_r, zeros)


def _bn(p, g, b):
  m = jnp.mean(p, axis=0, keepdims=True)
  v = jnp.mean((p - m) * (p - m), axis=0, keepdims=True)
  return (p - m) * lax.rsqrt(v + 1e-5) * g + b


def _dot(a, b):
  return jnp.dot(a, b, preferred_element_type=jnp.float32)


def _tc(body, out_shape, *args):
  return pl.pallas_call(
      body, out_shape=jax.ShapeDtypeStruct(out_shape, jnp.float32))(*args)


def _pre_stem(x, sU):
  def body(x_ref, u_ref, o_ref):
    o_ref[...] = _dot(x_ref[...], u_ref[...])
  return _tc(body, (N, sU.shape[1]), x, sU)


def _stem(x, ag, sW, sb, sg, sB):
  def body(x_ref, ag_ref, w_ref, b_ref, g_ref, bb_ref, o_ref):
    p = _dot(x_ref[...], w_ref[...]) + ag_ref[0] + ag_ref[1] + b_ref[...]
    o_ref[...] = jnp.maximum(_bn(p, g_ref[...], bb_ref[...]), 0.0)
  return _tc(body, (N, sW.shape[1]), x, ag, sW, sb, sg, sB)


def _conv_a(h, ag, W, U, b, g, B):
  def body(h_ref, ag_ref, w_ref, u_ref, b_ref, g_ref, bb_ref, o_ref):
    p = (_dot(h_ref[...], w_ref[...])
         + _dot(ag_ref[0] + ag_ref[1], u_ref[...]) + b_ref[...])
    o_ref[...] = jnp.maximum(_bn(p, g_ref[...], bb_ref[...]), 0.0)
  return _tc(body, (N, W.shape[1]), h, ag, W, U, b, g, B)


def _conv_b_res(ha, ag, hin, W, U, b, g, B, Wd, gd, Bd):
  def body(ha_ref, ag_ref, hin_ref, w_ref, u_ref, b_ref, g_ref, bb_ref,
           wd_ref, gd_ref, bd_ref, o_ref):
    p = (_dot(ha_ref[...], w_ref[...])
         + _dot(ag_ref[0] + ag_ref[1], u_ref[...]) + b_ref[...])
    hb = _bn(p, g_ref[...], bb_ref[...])
    sc = _bn(_dot(hin_ref[...], wd_ref[...]), gd_ref[...], bd_ref[...])
    o_ref[...] = jnp.maximum(hb + sc, 0.0)
  return _tc(body, (N, W.shape[1]), ha, ag, hin, W, U, b, g, B, Wd, gd, Bd)


def _final(ha, ag, hin, W, U, b, g, B, Wd, gd, Bd, lW1, lb1, lW2, lb2):
  def body(ha_ref, ag_ref, hin_ref, w_ref, u_ref, b_ref, g_ref, bb_ref,
           wd_ref, gd_ref, bd_ref, w1_ref, b1_ref, w2_ref, b2_ref, o_ref):
    p = (_dot(ha_ref[...], w_ref[...])
         + _dot(ag_ref[0] + ag_ref[1], u_ref[...]) + b_ref[...])
    hb = _bn(p, g_ref[...], bb_ref[...])
    sc = _bn(_dot(hin_ref[...], wd_ref[...]), gd_ref[...], bd_ref[...])
    h2 = jnp.maximum(hb + sc, 0.0)
    z = _dot(h2, w1_ref[...]) + b1_ref[...]
    z = jnp.where(z > 0, z, 0.1 * z)
    o_ref[...] = jax.nn.sigmoid(_dot(z, w2_ref[...]) + b2_ref[...])
  return _tc(body, (N, lW2.shape[1]), ha, ag, hin, W, U, b, g, B,
             Wd, gd, Bd, lW1, lb1, lW2, lb2)


def kernel(x, edge_index, sW, sU, sb, sg, sB,
           a1W, a1U, a1b, a1g, a1B, b1W, b1U, b1b, b1g, b1B, d1W, d1g, d1B,
           a2W, a2U, a2b, a2g, a2B, b2W, b2U, b2b, b2g, b2B, d2W, d2g, d2B,
           lW1, lb1, lW2, lb2):
  ei = edge_index.astype(jnp.int32)
  src_r = ei[0].reshape(NW, NCH, CHUNK)
  dst_r = ei[1].reshape(NW, NCH, CHUNK)
  z32 = jnp.zeros((N, 32), jnp.float32)
  z64 = jnp.zeros((N, 64), jnp.float32)
  z128 = jnp.zeros((N, 128), jnp.float32)

  pre0 = _pre_stem(x, sU)
  ag0 = _spmm_sc(pre0, src_r, dst_r, z32, 32)
  h0 = _stem(x, ag0, sW, sb, sg, sB)

  ag1 = _spmm_sc(h0, src_r, dst_r, z32, 32)
  ha1 = _conv_a(h0, ag1, a1W, a1U, a1b, a1g, a1B)
  ag2 = _spmm_sc(ha1, src_r, dst_r, z64, 64)
  h1 = _conv_b_res(ha1, ag2, h0, b1W, b1U, b1b, b1g, b1B,
                   d1W, d1g, d1B)

  ag3 = _spmm_sc(h1, src_r, dst_r, z64, 64)
  ha2 = _conv_a(h1, ag3, a2W, a2U, a2b, a2g, a2B)
  ag4 = _spmm_sc(ha2, src_r, dst_r, z128, 128)
  return _final(ha2, ag4, h1, b2W, b2U, b2b, b2g, b2B,
                d2W, d2g, d2B, lW1, lb1, lW2, lb2)


# speedup vs baseline: 10.3048x; 1.1895x over previous
import functools

import jax
import jax.numpy as jnp
from jax import lax
from jax.experimental import pallas as pl
from jax.experimental.pallas import tpu as pltpu
from jax.experimental.pallas import tpu_sc as plsc

N = 10000
E = 320000
NCORE = 2
NSUB = 16
NW = NCORE * NSUB
EW = E // NW
CHUNK = 80
NCH = EW // CHUNK
ROW_STRIDE = 624
ROW_SLAB = 640


def _spmm_sc(h, src_r, dst_r, zeros, C):
  mesh = plsc.VectorSubcoreMesh(core_axis_name="c", subcore_axis_name="s")

  @functools.partial(
      pl.kernel,
      out_type=jax.ShapeDtypeStruct((NCORE, N, C), jnp.float32),
      mesh=mesh,
      compiler_params=pltpu.CompilerParams(use_tc_tiling_on_sc=False),
      scratch_types=[
          pltpu.VMEM((NCH, CHUNK), jnp.int32),
          pltpu.VMEM((NCH, CHUNK), jnp.int32),
          pltpu.VMEM((CHUNK, C), jnp.float32),
          pltpu.VMEM((CHUNK, C), jnp.float32),
          pltpu.VMEM_SHARED((N, C), jnp.float32),
          pltpu.SemaphoreType.DMA,
          pltpu.SemaphoreType.DMA,
          pltpu.SemaphoreType.DMA,
          pltpu.SemaphoreType.DMA,
      ],
  )
  def spmm(h_hbm, src_hbm, dst_hbm, z_hbm, out_hbm,
           src_v, dst_v, rows0, rows1, agg_sh, g0, g1, s0, s1):
    cid = lax.axis_index("c")
    sid = lax.axis_index("s")
    wid = sid * NCORE + cid
    r0 = sid * ROW_STRIDE
    pltpu.sync_copy(z_hbm.at[pl.ds(r0, ROW_SLAB)],
                    agg_sh.at[pl.ds(r0, ROW_SLAB)])
    pltpu.sync_copy(src_hbm.at[wid], src_v)
    pltpu.sync_copy(dst_hbm.at[wid], dst_v)
    plsc.subcore_barrier()

    rows = (rows0, rows1)
    gsem = (g0, g1)
    ssem = (s0, s1)

    def wait_gather(b):
      pltpu.make_async_copy(h_hbm.at[pl.ds(0, CHUNK)], rows[b], gsem[b]).wait()

    def wait_scatter(b):
      pltpu.make_async_copy(rows[b], agg_sh.at[pl.ds(0, CHUNK)],
                            ssem[b]).wait()

    pltpu.async_copy(h_hbm.at[src_v.at[0]], rows0, g0)

    @pl.loop(0, NCH - 1, step=2)
    def pair(j):
      for b in (0, 1):
        i = j + b
        wait_gather(b)

        @pl.when(i > 0)
        def _():
          wait_scatter(1 - b)

        pltpu.async_copy(h_hbm.at[src_v.at[i + 1]], rows[1 - b],
                         gsem[1 - b])
        pltpu.async_copy(rows[b], agg_sh.at[dst_v.at[i]], ssem[b], add=True)

    wait_gather(0)
    wait_scatter(1)
    pltpu.sync_copy(rows0, agg_sh.at[dst_v.at[NCH - 1]], add=True)
    plsc.subcore_barrier()
    pltpu.sync_copy(agg_sh.at[pl.ds(r0, ROW_SLAB)],
                    out_hbm.at[cid, pl.ds(r0, ROW_SLAB)])

  return spmm(h, src_r, dst_r, zeros)


def _bn(p, g, b):
  m = jnp.mean(p, axis=0, keepdims=True)
  v = jnp.mean((p - m) * (p - m), axis=0, keepdims=True)
  return (p - m) * lax.rsqrt(v + 1e-5) * g + b


def _dot(a, b):
  return jnp.dot(a, b, preferred_element_type=jnp.float32)


def _tc(body, out_shape, *args):
  return pl.pallas_call(
      body, out_shape=jax.ShapeDtypeStruct(out_shape, jnp.float32))(*args)


def _pre_stem(x, sU):
  def body(x_ref, u_ref, o_ref):
    o_ref[...] = _dot(x_ref[...], u_ref[...])
  return _tc(body, (N, sU.shape[1]), x, sU)


def _stem(x, ag, sW, sb, sg, sB):
  def body(x_ref, ag_ref, w_ref, b_ref, g_ref, bb_ref, o_ref):
    p = _dot(x_ref[...], w_ref[...]) + ag_ref[0] + ag_ref[1] + b_ref[...]
    o_ref[...] = jnp.maximum(_bn(p, g_ref[...], bb_ref[...]), 0.0)
  return _tc(body, (N, sW.shape[1]), x, ag, sW, sb, sg, sB)


def _conv_a(h, ag, W, U, b, g, B):
  def body(h_ref, ag_ref, w_ref, u_ref, b_ref, g_ref, bb_ref, o_ref):
    p = (_dot(h_ref[...], w_ref[...])
         + _dot(ag_ref[0] + ag_ref[1], u_ref[...]) + b_ref[...])
    o_ref[...] = jnp.maximum(_bn(p, g_ref[...], bb_ref[...]), 0.0)
  return _tc(body, (N, W.shape[1]), h, ag, W, U, b, g, B)


def _conv_b_res(ha, ag, hin, W, U, b, g, B, Wd, gd, Bd):
  def body(ha_ref, ag_ref, hin_ref, w_ref, u_ref, b_ref, g_ref, bb_ref,
           wd_ref, gd_ref, bd_ref, o_ref):
    p = (_dot(ha_ref[...], w_ref[...])
         + _dot(ag_ref[0] + ag_ref[1], u_ref[...]) + b_ref[...])
    hb = _bn(p, g_ref[...], bb_ref[...])
    sc = _bn(_dot(hin_ref[...], wd_ref[...]), gd_ref[...], bd_ref[...])
    o_ref[...] = jnp.maximum(hb + sc, 0.0)
  return _tc(body, (N, W.shape[1]), ha, ag, hin, W, U, b, g, B, Wd, gd, Bd)


def _final(ha, ag, hin, W, U, b, g, B, Wd, gd, Bd, lW1, lb1, lW2, lb2):
  def body(ha_ref, ag_ref, hin_ref, w_ref, u_ref, b_ref, g_ref, bb_ref,
           wd_ref, gd_ref, bd_ref, w1_ref, b1_ref, w2_ref, b2_ref, o_ref):
    p = (_dot(ha_ref[...], w_ref[...])
         + _dot(ag_ref[0] + ag_ref[1], u_ref[...]) + b_ref[...])
    hb = _bn(p, g_ref[...], bb_ref[...])
    sc = _bn(_dot(hin_ref[...], wd_ref[...]), gd_ref[...], bd_ref[...])
    h2 = jnp.maximum(hb + sc, 0.0)
    z = _dot(h2, w1_ref[...]) + b1_ref[...]
    z = jnp.where(z > 0, z, 0.1 * z)
    o_ref[...] = jax.nn.sigmoid(_dot(z, w2_ref[...]) + b2_ref[...])
  return _tc(body, (N, lW2.shape[1]), ha, ag, hin, W, U, b, g, B,
             Wd, gd, Bd, lW1, lb1, lW2, lb2)


def kernel(x, edge_index, sW, sU, sb, sg, sB,
           a1W, a1U, a1b, a1g, a1B, b1W, b1U, b1b, b1g, b1B, d1W, d1g, d1B,
           a2W, a2U, a2b, a2g, a2B, b2W, b2U, b2b, b2g, b2B, d2W, d2g, d2B,
           lW1, lb1, lW2, lb2):
  ei = edge_index.astype(jnp.int32)
  src_r = ei[0].reshape(NW, NCH, CHUNK)
  dst_r = ei[1].reshape(NW, NCH, CHUNK)
  z32 = jnp.zeros((N, 32), jnp.float32)
  z64 = jnp.zeros((N, 64), jnp.float32)
  z128 = jnp.zeros((N, 128), jnp.float32)

  pre0 = _pre_stem(x, sU)
  ag0 = _spmm_sc(pre0, src_r, dst_r, z32, 32)
  h0 = _stem(x, ag0, sW, sb, sg, sB)

  ag1 = _spmm_sc(h0, src_r, dst_r, z32, 32)
  ha1 = _conv_a(h0, ag1, a1W, a1U, a1b, a1g, a1B)
  ag2 = _spmm_sc(ha1, src_r, dst_r, z64, 64)
  h1 = _conv_b_res(ha1, ag2, h0, b1W, b1U, b1b, b1g, b1B,
                   d1W, d1g, d1B)

  ag3 = _spmm_sc(h1, src_r, dst_r, z64, 64)
  ha2 = _conv_a(h1, ag3, a2W, a2U, a2b, a2g, a2B)
  ag4 = _spmm_sc(ha2, src_r, dst_r, z128, 128)
  return _final(ha2, ag4, h1, b2W, b2U, b2b, b2g, b2B,
                d2W, d2g, d2B, lW1, lb1, lW2, lb2)
